# baseline (speedup 1.0000x reference)
import functools

import jax
import jax.numpy as jnp
from jax import lax
from jax.experimental import pallas as pl
from jax.experimental.pallas import tpu as pltpu
from jax.experimental.pallas import tpu_sc as plsc

N_NODES = 10000
N_PAD = 10240
E = 320000
D = 128
AUG = 128
NG = 64
NC = 2
NS = 16
NW = NC * NS
CHUNK = 128
EPT0 = 112
EPT1 = 46
E_PAD = NS * (EPT0 + EPT1) * CHUNK
X_PAD = 12288
RPT = N_PAD // NS
BLK = 512
GRID = N_PAD // BLK


def _wid():
    return lax.axis_index("c") * NS + lax.axis_index("s")


@functools.lru_cache(maxsize=None)
def _sc_kernels():
    mesh = plsc.VectorSubcoreMesh(
        core_axis_name="c", subcore_axis_name="s",
        num_cores=NC, num_subcores=NS)

    @functools.partial(
        pl.kernel,
        out_type=jax.ShapeDtypeStruct((X_PAD, D), jnp.float32),
        mesh=mesh,
        scratch_types=[
            pltpu.VMEM((CHUNK,), jnp.int32),
            pltpu.VMEM((CHUNK, D), jnp.float32),
            pltpu.SemaphoreType.DMA,
        ],
    )
    def _gather(table, idx, out, idx_v, rows_v, sem):
        base = _wid() * ((X_PAD // NW // CHUNK) * CHUNK)

        def body(j, carry):
            off = pl.multiple_of(base + j * CHUNK, CHUNK)
            pltpu.sync_copy(idx.at[pl.ds(off, CHUNK)], idx_v)
            pltpu.async_copy(table.at[idx_v], rows_v, sem).wait()
            pltpu.sync_copy(rows_v, out.at[pl.ds(off, CHUNK)])
            return carry

        lax.fori_loop(0, X_PAD // NW // CHUNK, body, 0)

    @functools.partial(
        pl.kernel,
        out_type=(
            jax.ShapeDtypeStruct((N_PAD, D), jnp.float32),
            jax.ShapeDtypeStruct((N_PAD, D), jnp.float32),
        ),
        mesh=mesh,
        scratch_types=[
            [pltpu.VMEM((CHUNK,), jnp.int32)] * 2,
            [pltpu.VMEM((CHUNK,), jnp.int32)] * 2,
            [pltpu.VMEM((CHUNK, D), jnp.float32)] * 2,
            pltpu.VMEM_SHARED((N_PAD, D), jnp.float32),
            pltpu.SemaphoreType.DMA,
            pltpu.SemaphoreType.DMA,
        ],
    )
    def _agg(table, src, dst, zeros, out0, out1,
             src_v, dst_v, rows_v, acc, gsem, ssem):
        c = lax.axis_index("c")
        s = lax.axis_index("s")
        r0 = pl.multiple_of(s * RPT, RPT)
        pltpu.sync_copy(zeros.at[pl.ds(r0, RPT)], acc.at[pl.ds(r0, RPT)])
        plsc.subcore_barrier()
        base = jnp.where(c == 0, s * EPT0,
                         NS * EPT0 + s * EPT1) * CHUNK
        ept = jnp.where(c == 0, EPT0, EPT1)

        def body(g, carry):
            j0 = g * 2
            off0 = pl.multiple_of(base + j0 * CHUNK, CHUNK)
            off1 = pl.multiple_of(base + (j0 + 1) * CHUNK, CHUNK)
            pltpu.sync_copy(src.at[pl.ds(off0, CHUNK)], src_v[0])
            pltpu.sync_copy(dst.at[pl.ds(off0, CHUNK)], dst_v[0])
            g0 = pltpu.async_copy(table.at[src_v[0]], rows_v[0], gsem)
            pltpu.sync_copy(src.at[pl.ds(off1, CHUNK)], src_v[1])
            pltpu.sync_copy(dst.at[pl.ds(off1, CHUNK)], dst_v[1])
            g0.wait()
            s0 = pltpu.async_copy(rows_v[0], acc.at[dst_v[0]], ssem, add=True)
            pltpu.async_copy(table.at[src_v[1]], rows_v[1], gsem).wait()
            s0.wait()
            pltpu.sync_copy(rows_v[1], acc.at[dst_v[1]], add=True)
            return carry

        lax.fori_loop(0, ept // 2, body, 0)
        plsc.subcore_barrier()

        @pl.when(c == 0)
        def _():
            pltpu.sync_copy(acc.at[pl.ds(r0, RPT)], out0.at[pl.ds(r0, RPT)])

        @pl.when(c == 1)
        def _():
            pltpu.sync_copy(acc.at[pl.ds(r0, RPT)], out1.at[pl.ds(r0, RPT)])

    return _gather, _agg


def _mm_body(h_ref, w_ref, o_ref):
    o_ref[...] = jnp.dot(h_ref[...], w_ref[...],
                         preferred_element_type=jnp.float32)


def _mm(h, w):
    return pl.pallas_call(
        _mm_body,
        out_shape=jax.ShapeDtypeStruct((N_PAD, D), jnp.float32),
        grid=(GRID,),
        in_specs=[
            pl.BlockSpec((BLK, D), lambda i: (i, 0)),
            pl.BlockSpec((D, D), lambda i: (0, 0)),
        ],
        out_specs=pl.BlockSpec((BLK, D), lambda i: (i, 0)),
    )(h, w)


def _hidden(p0, p1, ea0, ea1, wn, we, bias):
    agg = jnp.dot(p0[...] + p1[...], wn[...],
                  preferred_element_type=jnp.float32)
    e_blk = ea0[...] + ea1[...]
    esum = e_blk[:, :32]
    deg = e_blk[:, 32:33]
    aggf = agg + jnp.dot(esum, we[...], preferred_element_type=jnp.float32)
    return jnp.maximum(aggf / jnp.maximum(deg, 1.0) + bias[...], 0.0)


def _boundary_body(p0, p1, ea0, ea1, wn, we, bias, o_ref):
    o_ref[...] = _hidden(p0, p1, ea0, ea1, wn, we, bias)


def _boundary(p0, p1, ea0, ea1, wn, we, bias):
    return pl.pallas_call(
        _boundary_body,
        out_shape=jax.ShapeDtypeStruct((N_PAD, D), jnp.float32),
        grid=(GRID,),
        in_specs=[
            pl.BlockSpec((BLK, D), lambda i: (i, 0)),
            pl.BlockSpec((BLK, D), lambda i: (i, 0)),
            pl.BlockSpec((BLK, AUG), lambda i: (i, 0)),
            pl.BlockSpec((BLK, AUG), lambda i: (i, 0)),
            pl.BlockSpec((D, D), lambda i: (0, 0)),
            pl.BlockSpec((32, D), lambda i: (0, 0)),
            pl.BlockSpec((1, D), lambda i: (0, 0)),
        ],
        out_specs=pl.BlockSpec((BLK, D), lambda i: (i, 0)),
    )(p0, p1, ea0, ea1, wn, we, bias)


def _final_body(p0, p1, ea0, ea1, wn, we, bias, oh_ref, o_ref):
    h = _hidden(p0, p1, ea0, ea1, wn, we, bias)

    @pl.when(pl.program_id(0) == 0)
    def _():
        o_ref[...] = jnp.zeros_like(o_ref)

    oh = oh_ref[...]
    rows = [jnp.max(h * oh[:, g:g + 1], axis=0) for g in range(NG)]
    o_ref[...] = jnp.maximum(o_ref[...], jnp.stack(rows, axis=0))


def _final(p0, p1, ea0, ea1, wn, we, bias, oh):
    return pl.pallas_call(
        _final_body,
        out_shape=jax.ShapeDtypeStruct((NG, D), jnp.float32),
        grid=(GRID,),
        in_specs=[
            pl.BlockSpec((BLK, D), lambda i: (i, 0)),
            pl.BlockSpec((BLK, D), lambda i: (i, 0)),
            pl.BlockSpec((BLK, AUG), lambda i: (i, 0)),
            pl.BlockSpec((BLK, AUG), lambda i: (i, 0)),
            pl.BlockSpec((D, D), lambda i: (0, 0)),
            pl.BlockSpec((32, D), lambda i: (0, 0)),
            pl.BlockSpec((1, D), lambda i: (0, 0)),
            pl.BlockSpec((BLK, NG), lambda i: (i, 0)),
        ],
        out_specs=pl.BlockSpec((NG, D), lambda i: (0, 0)),
    )(p0, p1, ea0, ea1, wn, we, bias, oh)


def kernel(x, edge_attr, edge_index, batch, embed, edge_embed, W_node, W_edge, b):
    x = x.astype(jnp.int32)
    ea = edge_attr.astype(jnp.int32)
    src = edge_index[0].astype(jnp.int32)
    dst = edge_index[1].astype(jnp.int32)
    bt = batch.astype(jnp.int32)

    pad_e = E_PAD - E
    x_pad = jnp.concatenate([x, jnp.zeros((X_PAD - N_NODES,), jnp.int32)])
    src_pad = jnp.concatenate([src, jnp.zeros((pad_e,), jnp.int32)])
    trash = N_NODES + (jnp.arange(pad_e, dtype=jnp.int32) % (N_PAD - N_NODES))
    dst_pad = jnp.concatenate([dst, trash])
    ea_pad = jnp.concatenate([ea, jnp.full((pad_e,), 200, jnp.int32)])

    aug = jnp.zeros((208, AUG), jnp.float32)
    aug = aug.at[:200, :32].set(edge_embed).at[:200, 32].set(1.0)

    zeros128 = jnp.zeros((N_PAD, D), jnp.float32)
    oh = jnp.concatenate(
        [jax.nn.one_hot(bt, NG, dtype=jnp.float32),
         jnp.zeros((N_PAD - N_NODES, NG), jnp.float32)], axis=0)

    _gather, _agg = _sc_kernels()
    h0 = _gather(embed, x_pad)
    ea0, ea1 = _agg(aug, ea_pad, dst_pad, zeros128)
    h = h0
    for l in range(3):
        p0, p1 = _agg(h, src_pad, dst_pad, zeros128)
        if l < 2:
            h = _boundary(p0, p1, ea0, ea1, W_node[l], W_edge[l], b[l][None])
        else:
            out = _final(p0, p1, ea0, ea1, W_node[2], W_edge[2], b[2][None], oh)
    return out


# speedup vs baseline: 1.0152x; 1.0152x over previous
import functools

import jax
import jax.numpy as jnp
from jax import lax
from jax.experimental import pallas as pl
from jax.experimental.pallas import tpu as pltpu
from jax.experimental.pallas import tpu_sc as plsc

N_NODES = 10000
N_PAD = 10240
E = 320000
D = 128
AUG = 128
NG = 64
NC = 2
NS = 16
NW = NC * NS
CHUNK = 128
EPT0 = 106
EPT1 = 52
E_PAD = NS * (EPT0 + EPT1) * CHUNK
X_PAD = 12288
RPT = N_PAD // NS
BLK = 512
GRID = N_PAD // BLK


def _wid():
    return lax.axis_index("c") * NS + lax.axis_index("s")


@functools.lru_cache(maxsize=None)
def _sc_kernels():
    mesh = plsc.VectorSubcoreMesh(
        core_axis_name="c", subcore_axis_name="s",
        num_cores=NC, num_subcores=NS)

    @functools.partial(
        pl.kernel,
        out_type=jax.ShapeDtypeStruct((X_PAD, D), jnp.float32),
        mesh=mesh,
        scratch_types=[
            pltpu.VMEM((CHUNK,), jnp.int32),
            pltpu.VMEM((CHUNK, D), jnp.float32),
            pltpu.SemaphoreType.DMA,
        ],
    )
    def _gather(table, idx, out, idx_v, rows_v, sem):
        base = _wid() * ((X_PAD // NW // CHUNK) * CHUNK)

        def body(j, carry):
            off = pl.multiple_of(base + j * CHUNK, CHUNK)
            pltpu.sync_copy(idx.at[pl.ds(off, CHUNK)], idx_v)
            pltpu.async_copy(table.at[idx_v], rows_v, sem).wait()
            pltpu.sync_copy(rows_v, out.at[pl.ds(off, CHUNK)])
            return carry

        lax.fori_loop(0, X_PAD // NW // CHUNK, body, 0)

    @functools.partial(
        pl.kernel,
        out_type=(
            jax.ShapeDtypeStruct((N_PAD, D), jnp.float32),
            jax.ShapeDtypeStruct((N_PAD, D), jnp.float32),
        ),
        mesh=mesh,
        scratch_types=[
            [pltpu.VMEM((CHUNK,), jnp.int32)] * 2,
            [pltpu.VMEM((CHUNK,), jnp.int32)] * 2,
            [pltpu.VMEM((CHUNK, D), jnp.float32)] * 2,
            pltpu.VMEM_SHARED((N_PAD, D), jnp.float32),
            pltpu.SemaphoreType.DMA,
            pltpu.SemaphoreType.DMA,
        ],
    )
    def _agg(table, src, dst, zeros, out0, out1,
             src_v, dst_v, rows_v, acc, gsem, ssem):
        c = lax.axis_index("c")
        s = lax.axis_index("s")
        r0 = pl.multiple_of(s * RPT, RPT)
        pltpu.sync_copy(zeros.at[pl.ds(r0, RPT)], acc.at[pl.ds(r0, RPT)])
        plsc.subcore_barrier()
        base = jnp.where(c == 0, s * EPT0,
                         NS * EPT0 + s * EPT1) * CHUNK
        ept = jnp.where(c == 0, EPT0, EPT1)

        def body(g, carry):
            j0 = g * 2
            off0 = pl.multiple_of(base + j0 * CHUNK, CHUNK)
            off1 = pl.multiple_of(base + (j0 + 1) * CHUNK, CHUNK)
            pltpu.sync_copy(src.at[pl.ds(off0, CHUNK)], src_v[0])
            pltpu.sync_copy(dst.at[pl.ds(off0, CHUNK)], dst_v[0])
            g0 = pltpu.async_copy(table.at[src_v[0]], rows_v[0], gsem)
            pltpu.sync_copy(src.at[pl.ds(off1, CHUNK)], src_v[1])
            pltpu.sync_copy(dst.at[pl.ds(off1, CHUNK)], dst_v[1])
            g0.wait()
            s0 = pltpu.async_copy(rows_v[0], acc.at[dst_v[0]], ssem, add=True)
            pltpu.async_copy(table.at[src_v[1]], rows_v[1], gsem).wait()
            s0.wait()
            pltpu.sync_copy(rows_v[1], acc.at[dst_v[1]], add=True)
            return carry

        lax.fori_loop(0, ept // 2, body, 0)
        plsc.subcore_barrier()

        @pl.when(c == 0)
        def _():
            pltpu.sync_copy(acc.at[pl.ds(r0, RPT)], out0.at[pl.ds(r0, RPT)])

        @pl.when(c == 1)
        def _():
            pltpu.sync_copy(acc.at[pl.ds(r0, RPT)], out1.at[pl.ds(r0, RPT)])

    return _gather, _agg


def _mm_body(h_ref, w_ref, o_ref):
    o_ref[...] = jnp.dot(h_ref[...], w_ref[...],
                         preferred_element_type=jnp.float32)


def _mm(h, w):
    return pl.pallas_call(
        _mm_body,
        out_shape=jax.ShapeDtypeStruct((N_PAD, D), jnp.float32),
        grid=(GRID,),
        in_specs=[
            pl.BlockSpec((BLK, D), lambda i: (i, 0)),
            pl.BlockSpec((D, D), lambda i: (0, 0)),
        ],
        out_specs=pl.BlockSpec((BLK, D), lambda i: (i, 0)),
    )(h, w)


def _hidden(p0, p1, ea0, ea1, wn, we, bias):
    agg = jnp.dot(p0[...] + p1[...], wn[...],
                  preferred_element_type=jnp.float32)
    e_blk = ea0[...] + ea1[...]
    esum = e_blk[:, :32]
    deg = e_blk[:, 32:33]
    aggf = agg + jnp.dot(esum, we[...], preferred_element_type=jnp.float32)
    return jnp.maximum(aggf / jnp.maximum(deg, 1.0) + bias[...], 0.0)


def _boundary_body(p0, p1, ea0, ea1, wn, we, bias, o_ref):
    o_ref[...] = _hidden(p0, p1, ea0, ea1, wn, we, bias)


def _boundary(p0, p1, ea0, ea1, wn, we, bias):
    return pl.pallas_call(
        _boundary_body,
        out_shape=jax.ShapeDtypeStruct((N_PAD, D), jnp.float32),
        grid=(GRID,),
        in_specs=[
            pl.BlockSpec((BLK, D), lambda i: (i, 0)),
            pl.BlockSpec((BLK, D), lambda i: (i, 0)),
            pl.BlockSpec((BLK, AUG), lambda i: (i, 0)),
            pl.BlockSpec((BLK, AUG), lambda i: (i, 0)),
            pl.BlockSpec((D, D), lambda i: (0, 0)),
            pl.BlockSpec((32, D), lambda i: (0, 0)),
            pl.BlockSpec((1, D), lambda i: (0, 0)),
        ],
        out_specs=pl.BlockSpec((BLK, D), lambda i: (i, 0)),
    )(p0, p1, ea0, ea1, wn, we, bias)


def _final_body(p0, p1, ea0, ea1, wn, we, bias, oh_ref, o_ref):
    h = _hidden(p0, p1, ea0, ea1, wn, we, bias)

    @pl.when(pl.program_id(0) == 0)
    def _():
        o_ref[...] = jnp.zeros_like(o_ref)

    oh = oh_ref[...]
    rows = [jnp.max(h * oh[:, g:g + 1], axis=0) for g in range(NG)]
    o_ref[...] = jnp.maximum(o_ref[...], jnp.stack(rows, axis=0))


def _final(p0, p1, ea0, ea1, wn, we, bias, oh):
    return pl.pallas_call(
        _final_body,
        out_shape=jax.ShapeDtypeStruct((NG, D), jnp.float32),
        grid=(GRID,),
        in_specs=[
            pl.BlockSpec((BLK, D), lambda i: (i, 0)),
            pl.BlockSpec((BLK, D), lambda i: (i, 0)),
            pl.BlockSpec((BLK, AUG), lambda i: (i, 0)),
            pl.BlockSpec((BLK, AUG), lambda i: (i, 0)),
            pl.BlockSpec((D, D), lambda i: (0, 0)),
            pl.BlockSpec((32, D), lambda i: (0, 0)),
            pl.BlockSpec((1, D), lambda i: (0, 0)),
            pl.BlockSpec((BLK, NG), lambda i: (i, 0)),
        ],
        out_specs=pl.BlockSpec((NG, D), lambda i: (0, 0)),
    )(p0, p1, ea0, ea1, wn, we, bias, oh)


def kernel(x, edge_attr, edge_index, batch, embed, edge_embed, W_node, W_edge, b):
    x = x.astype(jnp.int32)
    ea = edge_attr.astype(jnp.int32)
    src = edge_index[0].astype(jnp.int32)
    dst = edge_index[1].astype(jnp.int32)
    bt = batch.astype(jnp.int32)

    pad_e = E_PAD - E
    x_pad = jnp.concatenate([x, jnp.zeros((X_PAD - N_NODES,), jnp.int32)])
    src_pad = jnp.concatenate([src, jnp.zeros((pad_e,), jnp.int32)])
    trash = N_NODES + (jnp.arange(pad_e, dtype=jnp.int32) % (N_PAD - N_NODES))
    dst_pad = jnp.concatenate([dst, trash])
    ea_pad = jnp.concatenate([ea, jnp.full((pad_e,), 200, jnp.int32)])

    aug = jnp.zeros((208, AUG), jnp.float32)
    aug = aug.at[:200, :32].set(edge_embed).at[:200, 32].set(1.0)

    zeros128 = jnp.zeros((N_PAD, D), jnp.float32)
    oh = jnp.concatenate(
        [jax.nn.one_hot(bt, NG, dtype=jnp.float32),
         jnp.zeros((N_PAD - N_NODES, NG), jnp.float32)], axis=0)

    _gather, _agg = _sc_kernels()
    h0 = _gather(embed, x_pad)
    ea0, ea1 = _agg(aug, ea_pad, dst_pad, zeros128)
    h = h0
    for l in range(3):
        p0, p1 = _agg(h, src_pad, dst_pad, zeros128)
        if l < 2:
            h = _boundary(p0, p1, ea0, ea1, W_node[l], W_edge[l], b[l][None])
        else:
            out = _final(p0, p1, ea0, ea1, W_node[2], W_edge[2], b[2][None], oh)
    return out
